# baseline (speedup 1.0000x reference)
import functools

import jax
import jax.numpy as jnp
from jax import lax
from jax.experimental import pallas as pl
from jax.experimental.pallas import tpu as pltpu
from jax.experimental.pallas import tpu_sc as plsc

B = 16384
D = 8
N = 1000000
CB = 2048
SUBS = 16
NBLK = (N + SUBS * CB - 1) // (SUBS * CB)
NR = NBLK * SUBS * 128
NC = 2
NS = 16
NW = NC * NS
BPW = B // NW
CHUNK = 128
NCHUNK = BPW // CHUNK


def _repack_body(t0_ref, t1_ref, o0_ref, o1_ref):
    for t_ref, out_ref in ((t0_ref, o0_ref), (t1_ref, o1_ref)):
        x = t_ref[...]
        for sub in range(SUBS):
            xs = x[:, sub * CB:(sub + 1) * CB]
            x128 = jnp.concatenate(
                [xs[:, g * 128:(g + 1) * 128] for g in range(16)], axis=0)
            out_ref[pl.ds(sub * 128, 128), :] = x128.T


def _pack2(t0, t1):
    return pl.pallas_call(
        _repack_body,
        grid=(NBLK,),
        in_specs=[pl.BlockSpec((D, SUBS * CB), lambda i: (0, i))] * 2,
        out_specs=[pl.BlockSpec((SUBS * 128, 128), lambda i: (i, 0))] * 2,
        out_shape=[jax.ShapeDtypeStruct((NR, 128), jnp.float32)] * 2,
    )(t0.T, t1.T)


def _sc_gather_body(idx_hbm, p_a, p_b, o_a, o_b,
                    row_v, lane_v, gbuf, stage, sem):
    wid = lax.axis_index("s") * NC + lax.axis_index("c")
    base = wid * BPW
    lanes = lax.iota(jnp.int32, 16)
    half = lanes >> 3
    fvec = lanes & 7

    for j in range(NCHUNK):
        pltpu.sync_copy(idx_hbm.at[pl.ds(base + j * CHUNK, CHUNK)], row_v.at[j])

    def split(q, _):
        j = q // (CHUNK // 16)
        sl = pl.ds((q % (CHUNK // 16)) * 16, 16)
        v = row_v.at[j][sl]
        lane_v.at[j][sl] = ((v >> 7) & 15) * 8
        row_v.at[j][sl] = ((v >> 11) << 7) | (v & 127)
        return 0

    lax.fori_loop(0, BPW // 16, split, 0)

    for p_hbm, o_hbm in ((p_a, o_a), (p_b, o_b)):
        def chunk_body(j, _):
            pltpu.async_copy(p_hbm.at[row_v.at[j]], gbuf, sem).wait()
            for p in range(CHUNK // 2):
                loc = 2 * p + half
                lb = plsc.load_gather(lane_v, [jnp.full((16,), j, jnp.int32), loc])
                vals = plsc.load_gather(gbuf, [loc, lb + fvec])
                plsc.store_scatter(stage, [loc, fvec], vals)
            pltpu.sync_copy(stage, o_hbm.at[pl.ds(base + j * CHUNK, CHUNK)])
            return 0

        lax.fori_loop(0, NCHUNK, chunk_body, 0)


_sc_gather = functools.partial(
    pl.kernel,
    out_type=[jax.ShapeDtypeStruct((B, 128), jnp.float32)] * 2,
    mesh=plsc.VectorSubcoreMesh(core_axis_name="c", subcore_axis_name="s"),
    compiler_params=pltpu.CompilerParams(
        use_tc_tiling_on_sc=False, needs_layout_passes=False),
    scratch_types=[
        pltpu.VMEM((NCHUNK, CHUNK), jnp.int32),
        pltpu.VMEM((NCHUNK, CHUNK), jnp.int32),
        pltpu.VMEM((CHUNK, 128), jnp.float32),
        pltpu.VMEM((CHUNK, 128), jnp.float32),
        pltpu.SemaphoreType.DMA,
    ],
)(_sc_gather_body)


BLK = 2048


def _tc_mlp_body(u_mlp, i_mlp, u_mf, i_mf,
                 w0u, w0i, b0, w1t, b1, w2t, b2, wa_mlp, wa_mf, ba,
                 out):
    xu = u_mlp[...][:, :D]
    xi = i_mlp[...][:, :D]
    h = xu @ w0u[...] + xi @ w0i[...] + b0[...]
    h = jnp.maximum(h, 0.0)
    h = jnp.maximum(h @ w1t[...] + b1[...], 0.0)
    h = jnp.maximum(h @ w2t[...] + b2[...], 0.0)
    mf = u_mf[...][:, :D] * i_mf[...][:, :D]
    out[...] = h @ wa_mlp[...] + mf @ wa_mf[...] + ba[...]


def _full(shape):
    return pl.BlockSpec(shape, lambda i: (0,) * len(shape))


def kernel(user_indices, item_indices, emb_user_mlp, emb_item_mlp,
           emb_user_mf, emb_item_mf, W0, b0, W1, b1, W2, b2, Wa, ba):
    p_umlp, p_umf = _pack2(emb_user_mlp, emb_user_mf)
    g_umlp, g_umf = _sc_gather(user_indices, p_umlp, p_umf)
    p_imlp, p_imf = _pack2(emb_item_mlp, emb_item_mf)
    g_imlp, g_imf = _sc_gather(item_indices, p_imlp, p_imf)

    w0u = W0[:, :D].T
    w0i = W0[:, D:].T
    w1t = W1.T
    w2t = W2.T
    wa_mlp = Wa[:, :8].T
    wa_mf = Wa[:, 8:].T
    b0r = b0.reshape(1, -1)
    b1r = b1.reshape(1, -1)
    b2r = b2.reshape(1, -1)
    bar = ba.reshape(1, -1)

    out = pl.pallas_call(
        _tc_mlp_body,
        grid=(B // BLK,),
        in_specs=[
            pl.BlockSpec((BLK, 128), lambda i: (i, 0)),
            pl.BlockSpec((BLK, 128), lambda i: (i, 0)),
            pl.BlockSpec((BLK, 128), lambda i: (i, 0)),
            pl.BlockSpec((BLK, 128), lambda i: (i, 0)),
            _full((D, 32)), _full((D, 32)), _full((1, 32)),
            _full((32, 16)), _full((1, 16)),
            _full((16, 8)), _full((1, 8)),
            _full((8, 1)), _full((8, 1)), _full((1, 1)),
        ],
        out_specs=pl.BlockSpec((BLK, 1), lambda i: (i, 0)),
        out_shape=jax.ShapeDtypeStruct((B, 1), jnp.float32),
    )(g_umlp, g_imlp, g_umf, g_imf,
      w0u, w0i, b0r, w1t, b1r, w2t, b2r, wa_mlp, wa_mf, bar)
    return out


# speedup vs baseline: 1.0807x; 1.0807x over previous
import functools

import jax
import jax.numpy as jnp
from jax import lax
from jax.experimental import pallas as pl
from jax.experimental.pallas import tpu as pltpu
from jax.experimental.pallas import tpu_sc as plsc

B = 16384
D = 8
N = 1000000
CB = 2048
SUBS = 32
NBLK = (N + SUBS * CB - 1) // (SUBS * CB)
NR = NBLK * SUBS * 128
NC = 2
NS = 16
NW = NC * NS
BPW = B // NW
CHUNK = 128
NCHUNK = BPW // CHUNK


def _repack_body(t0_ref, t1_ref, o0_ref, o1_ref):
    for t_ref, out_ref in ((t0_ref, o0_ref), (t1_ref, o1_ref)):
        x = t_ref[...]
        for sub in range(SUBS):
            xs = x[:, sub * CB:(sub + 1) * CB]
            x128 = jnp.concatenate(
                [xs[:, g * 128:(g + 1) * 128] for g in range(16)], axis=0)
            out_ref[pl.ds(sub * 128, 128), :] = x128.T


def _pack2(t0, t1):
    return pl.pallas_call(
        _repack_body,
        grid=(NBLK,),
        in_specs=[pl.BlockSpec((D, SUBS * CB), lambda i: (0, i))] * 2,
        out_specs=[pl.BlockSpec((SUBS * 128, 128), lambda i: (i, 0))] * 2,
        out_shape=[jax.ShapeDtypeStruct((NR, 128), jnp.float32)] * 2,
    )(t0.T, t1.T)


def _sc_gather_body(idx_hbm, p_a, p_b, o_a, o_b,
                    row_v, lane_v, gbuf, stage, sem):
    wid = lax.axis_index("s") * NC + lax.axis_index("c")
    base = wid * BPW
    lanes = lax.iota(jnp.int32, 16)
    half = lanes >> 3
    fvec = lanes & 7

    for j in range(NCHUNK):
        pltpu.sync_copy(idx_hbm.at[pl.ds(base + j * CHUNK, CHUNK)], row_v.at[j])

    def split(q, _):
        j = q // (CHUNK // 16)
        sl = pl.ds((q % (CHUNK // 16)) * 16, 16)
        v = row_v.at[j][sl]
        lane_v.at[j][sl] = ((v >> 7) & 15) * 8
        row_v.at[j][sl] = ((v >> 11) << 7) | (v & 127)
        return 0

    lax.fori_loop(0, BPW // 16, split, 0)

    for p_hbm, o_hbm in ((p_a, o_a), (p_b, o_b)):
        def chunk_body(j, _):
            pltpu.async_copy(p_hbm.at[row_v.at[j]], gbuf, sem).wait()
            for p in range(CHUNK // 2):
                loc = 2 * p + half
                lb = plsc.load_gather(lane_v, [jnp.full((16,), j, jnp.int32), loc])
                vals = plsc.load_gather(gbuf, [loc, lb + fvec])
                plsc.store_scatter(stage, [loc, fvec], vals)
            pltpu.sync_copy(stage, o_hbm.at[pl.ds(base + j * CHUNK, CHUNK)])
            return 0

        lax.fori_loop(0, NCHUNK, chunk_body, 0)


_sc_gather = functools.partial(
    pl.kernel,
    out_type=[jax.ShapeDtypeStruct((B, 128), jnp.float32)] * 2,
    mesh=plsc.VectorSubcoreMesh(core_axis_name="c", subcore_axis_name="s"),
    compiler_params=pltpu.CompilerParams(
        use_tc_tiling_on_sc=False, needs_layout_passes=False),
    scratch_types=[
        pltpu.VMEM((NCHUNK, CHUNK), jnp.int32),
        pltpu.VMEM((NCHUNK, CHUNK), jnp.int32),
        pltpu.VMEM((CHUNK, 128), jnp.float32),
        pltpu.VMEM((CHUNK, 128), jnp.float32),
        pltpu.SemaphoreType.DMA,
    ],
)(_sc_gather_body)


BLK = 2048


def _tc_mlp_body(u_mlp, i_mlp, u_mf, i_mf,
                 w0u, w0i, b0, w1t, b1, w2t, b2, wa_mlp, wa_mf, ba,
                 out):
    xu = u_mlp[...][:, :D]
    xi = i_mlp[...][:, :D]
    h = xu @ w0u[...] + xi @ w0i[...] + b0[...]
    h = jnp.maximum(h, 0.0)
    h = jnp.maximum(h @ w1t[...] + b1[...], 0.0)
    h = jnp.maximum(h @ w2t[...] + b2[...], 0.0)
    mf = u_mf[...][:, :D] * i_mf[...][:, :D]
    out[...] = h @ wa_mlp[...] + mf @ wa_mf[...] + ba[...]


def _full(shape):
    return pl.BlockSpec(shape, lambda i: (0,) * len(shape))


def kernel(user_indices, item_indices, emb_user_mlp, emb_item_mlp,
           emb_user_mf, emb_item_mf, W0, b0, W1, b1, W2, b2, Wa, ba):
    p_umlp, p_umf = _pack2(emb_user_mlp, emb_user_mf)
    g_umlp, g_umf = _sc_gather(user_indices, p_umlp, p_umf)
    p_imlp, p_imf = _pack2(emb_item_mlp, emb_item_mf)
    g_imlp, g_imf = _sc_gather(item_indices, p_imlp, p_imf)

    w0u = W0[:, :D].T
    w0i = W0[:, D:].T
    w1t = W1.T
    w2t = W2.T
    wa_mlp = Wa[:, :8].T
    wa_mf = Wa[:, 8:].T
    b0r = b0.reshape(1, -1)
    b1r = b1.reshape(1, -1)
    b2r = b2.reshape(1, -1)
    bar = ba.reshape(1, -1)

    out = pl.pallas_call(
        _tc_mlp_body,
        grid=(B // BLK,),
        in_specs=[
            pl.BlockSpec((BLK, 128), lambda i: (i, 0)),
            pl.BlockSpec((BLK, 128), lambda i: (i, 0)),
            pl.BlockSpec((BLK, 128), lambda i: (i, 0)),
            pl.BlockSpec((BLK, 128), lambda i: (i, 0)),
            _full((D, 32)), _full((D, 32)), _full((1, 32)),
            _full((32, 16)), _full((1, 16)),
            _full((16, 8)), _full((1, 8)),
            _full((8, 1)), _full((8, 1)), _full((1, 1)),
        ],
        out_specs=pl.BlockSpec((BLK, 1), lambda i: (i, 0)),
        out_shape=jax.ShapeDtypeStruct((B, 1), jnp.float32),
    )(g_umlp, g_imlp, g_umf, g_imf,
      w0u, w0i, b0r, w1t, b1r, w2t, b2r, wa_mlp, wa_mf, bar)
    return out


# speedup vs baseline: 1.1083x; 1.0255x over previous
import functools

import jax
import jax.numpy as jnp
from jax import lax
from jax.experimental import pallas as pl
from jax.experimental.pallas import tpu as pltpu
from jax.experimental.pallas import tpu_sc as plsc

B = 16384
D = 8
N = 1000000
CB = 2048
SUBS = 32
NBLK = (N + SUBS * CB - 1) // (SUBS * CB)
NR = NBLK * SUBS * 128
NC = 2
NS = 16
NW = NC * NS
BPW = B // NW
CHUNK = 128
NCHUNK = BPW // CHUNK


def _repack_body(t0_ref, t1_ref, o0_ref, o1_ref):
    for t_ref, out_ref in ((t0_ref, o0_ref), (t1_ref, o1_ref)):
        x = t_ref[...]
        for sub in range(SUBS):
            xs = x[:, sub * CB:(sub + 1) * CB]
            x128 = jnp.concatenate(
                [xs[:, g * 128:(g + 1) * 128] for g in range(16)], axis=0)
            out_ref[pl.ds(sub * 128, 128), :] = x128.T


def _pack2(t0, t1):
    return pl.pallas_call(
        _repack_body,
        grid=(NBLK,),
        in_specs=[pl.BlockSpec((D, SUBS * CB), lambda i: (0, i))] * 2,
        out_specs=[pl.BlockSpec((SUBS * 128, 128), lambda i: (i, 0))] * 2,
        out_shape=[jax.ShapeDtypeStruct((NR, 128), jnp.float32)] * 2,
    )(t0.T, t1.T)


def _sc_gather_body(idx_hbm, p_a, p_b, o_a, o_b,
                    row_v, lane_v, gbuf, gbuf2, stage, sem):
    wid = lax.axis_index("s") * NC + lax.axis_index("c")
    base = wid * BPW
    lanes = lax.iota(jnp.int32, 16)
    half = lanes >> 3
    fvec = lanes & 7

    for j in range(NCHUNK):
        pltpu.sync_copy(idx_hbm.at[pl.ds(base + j * CHUNK, CHUNK)], row_v.at[j])

    def split(q, _):
        j = q // (CHUNK // 16)
        sl = pl.ds((q % (CHUNK // 16)) * 16, 16)
        v = row_v.at[j][sl]
        lane_v.at[j][sl] = ((v >> 7) & 15) * 8
        row_v.at[j][sl] = ((v >> 11) << 7) | (v & 127)
        return 0

    lax.fori_loop(0, BPW // 16, split, 0)

    steps = [(p, o, j) for p, o in ((p_a, o_a), (p_b, o_b))
             for j in range(NCHUNK)]
    bufs = (gbuf, gbuf2)
    handles = [pltpu.async_copy(steps[0][0].at[row_v.at[0]], bufs[0], sem)]
    for k, (p_hbm, o_hbm, j) in enumerate(steps):
        if k + 1 < len(steps):
            np_hbm, _, nj = steps[k + 1]
            handles.append(pltpu.async_copy(
                np_hbm.at[row_v.at[nj]], bufs[(k + 1) % 2], sem))
        handles[k].wait()
        gb = bufs[k % 2]
        for p in range(CHUNK // 2):
            loc = 2 * p + half
            lb = plsc.load_gather(lane_v, [jnp.full((16,), j, jnp.int32), loc])
            vals = plsc.load_gather(gb, [loc, lb + fvec])
            plsc.store_scatter(stage, [loc, fvec], vals)
        pltpu.sync_copy(stage, o_hbm.at[pl.ds(base + j * CHUNK, CHUNK)])


_sc_gather = functools.partial(
    pl.kernel,
    out_type=[jax.ShapeDtypeStruct((B, 128), jnp.float32)] * 2,
    mesh=plsc.VectorSubcoreMesh(core_axis_name="c", subcore_axis_name="s"),
    compiler_params=pltpu.CompilerParams(
        use_tc_tiling_on_sc=False, needs_layout_passes=False),
    scratch_types=[
        pltpu.VMEM((NCHUNK, CHUNK), jnp.int32),
        pltpu.VMEM((NCHUNK, CHUNK), jnp.int32),
        pltpu.VMEM((CHUNK, 128), jnp.float32),
        pltpu.VMEM((CHUNK, 128), jnp.float32),
        pltpu.VMEM((CHUNK, 128), jnp.float32),
        pltpu.SemaphoreType.DMA,
    ],
)(_sc_gather_body)


BLK = 2048


def _tc_mlp_body(u_mlp, i_mlp, u_mf, i_mf,
                 w0u, w0i, b0, w1t, b1, w2t, b2, wa_mlp, wa_mf, ba,
                 out):
    xu = u_mlp[...][:, :D]
    xi = i_mlp[...][:, :D]
    h = xu @ w0u[...] + xi @ w0i[...] + b0[...]
    h = jnp.maximum(h, 0.0)
    h = jnp.maximum(h @ w1t[...] + b1[...], 0.0)
    h = jnp.maximum(h @ w2t[...] + b2[...], 0.0)
    mf = u_mf[...][:, :D] * i_mf[...][:, :D]
    out[...] = h @ wa_mlp[...] + mf @ wa_mf[...] + ba[...]


def _full(shape):
    return pl.BlockSpec(shape, lambda i: (0,) * len(shape))


def kernel(user_indices, item_indices, emb_user_mlp, emb_item_mlp,
           emb_user_mf, emb_item_mf, W0, b0, W1, b1, W2, b2, Wa, ba):
    p_umlp, p_umf = _pack2(emb_user_mlp, emb_user_mf)
    g_umlp, g_umf = _sc_gather(user_indices, p_umlp, p_umf)
    p_imlp, p_imf = _pack2(emb_item_mlp, emb_item_mf)
    g_imlp, g_imf = _sc_gather(item_indices, p_imlp, p_imf)

    w0u = W0[:, :D].T
    w0i = W0[:, D:].T
    w1t = W1.T
    w2t = W2.T
    wa_mlp = Wa[:, :8].T
    wa_mf = Wa[:, 8:].T
    b0r = b0.reshape(1, -1)
    b1r = b1.reshape(1, -1)
    b2r = b2.reshape(1, -1)
    bar = ba.reshape(1, -1)

    out = pl.pallas_call(
        _tc_mlp_body,
        grid=(B // BLK,),
        in_specs=[
            pl.BlockSpec((BLK, 128), lambda i: (i, 0)),
            pl.BlockSpec((BLK, 128), lambda i: (i, 0)),
            pl.BlockSpec((BLK, 128), lambda i: (i, 0)),
            pl.BlockSpec((BLK, 128), lambda i: (i, 0)),
            _full((D, 32)), _full((D, 32)), _full((1, 32)),
            _full((32, 16)), _full((1, 16)),
            _full((16, 8)), _full((1, 8)),
            _full((8, 1)), _full((8, 1)), _full((1, 1)),
        ],
        out_specs=pl.BlockSpec((BLK, 1), lambda i: (i, 0)),
        out_shape=jax.ShapeDtypeStruct((B, 1), jnp.float32),
    )(g_umlp, g_imlp, g_umf, g_imf,
      w0u, w0i, b0r, w1t, b1r, w2t, b2r, wa_mlp, wa_mf, bar)
    return out
